# baseline (speedup 1.0000x reference)
import functools

import jax
import jax.numpy as jnp
from jax import lax
from jax.experimental import pallas as pl
from jax.experimental.pallas import tpu as pltpu
from jax.experimental.pallas import tpu_sc as plsc

N = 10000
E = 320000
HID = 128
EDIM = 16
NLAYERS = 7
VOCAB = 100
NFEAT = 9
HALF = 64
NC = 2
NS = 16
NG = 4
GK = 80
NPT = NG * GK
NPAD = NC * NS * NPT
EPT = E // NS
EK = 40
ECH = EPT // EK
NR0 = 624
NR1 = N - (NS - 1) * NR0
ZR = 16
EBR2 = 2560
EB2 = E // EBR2
F32 = jnp.float32


@functools.cache
def _sc_mesh():
    return plsc.VectorSubcoreMesh(core_axis_name="c", subcore_axis_name="s",
                                  num_cores=NC, num_subcores=NS)


def _atom_body(table_h, idx_h, out_h, idxv0, idxv1, gbuf0, gbuf1,
               idn0, idn1, idn2, idn3, zbuf,
               sem_z, sem_i0, sem_i1, sem_g0, sem_g1, sem_s0, sem_s1,
               acc_sh):
    c = lax.axis_index("c")
    s = lax.axis_index("s")
    base = (s * NC + c) * NPT
    idxv = [idxv0, idxv1]
    gbuf = [gbuf0, gbuf1]
    idn = [idn0, idn1, idn2, idn3]
    sem_i = [sem_i0, sem_i1]
    sem_g = [sem_g0, sem_g1]
    sem_s = [sem_s0, sem_s1]
    zeros16 = jnp.zeros((16,), F32)

    def zrow(i, carry):
        for k in range(HID // 16):
            zbuf[i, pl.ds(k * 16, 16)] = zeros16
        return carry

    lax.fori_loop(0, ZR, zrow, 0)
    for g in range(NG):
        for j in range(GK // 16):
            idn[g][pl.ds(j * 16, 16)] = (
                lax.iota(jnp.int32, 16) + (base + g * GK + j * 16))
    zs = [pltpu.async_copy(zbuf, acc_sh.at[pl.ds(base + i * ZR, ZR)], sem_z)
          for i in range(NPT // ZR)]
    for h in zs:
        h.wait()
    steps = [(f, g) for f in range(NFEAT) for g in range(NG)]
    hs = [None, None]
    for n, (f, g) in enumerate(steps):
        p = n % 2
        if hs[p] is not None:
            hs[p][1].wait()
        pltpu.async_copy(idx_h.at[pl.ds(f * NPAD + base + g * GK, GK)],
                         idxv[p], sem_i[p]).wait()
        gh = pltpu.async_copy(table_h.at[idxv[p]], gbuf[p], sem_g[p])
        gh.wait()
        sh = pltpu.async_copy(gbuf[p], acc_sh.at[idn[g]], sem_s[p], add=True)
        hs[p] = (gh, sh)
    for p in range(2):
        if hs[p] is not None:
            hs[p][1].wait()
    pltpu.sync_copy(acc_sh.at[pl.ds(base, NPT)], out_h.at[pl.ds(base, NPT)])


@functools.cache
def _atom_kernel():
    return pl.kernel(
        _atom_body,
        out_type=jax.ShapeDtypeStruct((NPAD, HID), F32),
        mesh=_sc_mesh(),
        scratch_types=(
            [pltpu.VMEM((GK,), jnp.int32)] * 2
            + [pltpu.VMEM((GK, HID), F32)] * 2
            + [pltpu.VMEM((GK,), jnp.int32)] * 4
            + [pltpu.VMEM((ZR, HID), F32)]
            + [pltpu.SemaphoreType.DMA] * 7
            + [pltpu.VMEM_SHARED((NPAD, HID), F32)]
        ),
    )


def _edge_body(src_h, dst_h, he_h, h1_h, b_h, nd_h,
               sidx0, sidx1, sidx2, didx0, didx1, didx2,
               heb0, heb1, heb2, gat0, gat1, gat2, tb0, tb1,
               bv, zbuf,
               sem_l0, sem_l1, sem_l2,
               sem_g0, sem_g1, sem_g2,
               sem_s0, sem_s1, sem_s2,
               acc_sh):
    sidx = [sidx0, sidx1, sidx2]
    didx = [didx0, didx1, didx2]
    heb = [heb0, heb1, heb2]
    gat = [gat0, gat1, gat2]
    tb = [tb0, tb1]
    sem_l = [sem_l0, sem_l1, sem_l2]
    sem_g = [sem_g0, sem_g1, sem_g2]
    sem_s = [sem_s0, sem_s1, sem_s2]
    c = lax.axis_index("c")
    s = lax.axis_index("s")
    zeros16 = jnp.zeros((16,), F32)

    def zrow(i, carry):
        for k in range(HID // 16):
            zbuf[i, pl.ds(k * 16, 16)] = zeros16
        return carry

    lax.fori_loop(0, ZR, zrow, 0)
    row0 = s * NR0
    zs = [pltpu.async_copy(zbuf, acc_sh.at[pl.ds(row0 + i * ZR, ZR)], sem_l0)
          for i in range(NR0 // ZR)]

    @pl.when(s == NS - 1)
    def _():
        pltpu.sync_copy(zbuf, acc_sh.at[pl.ds(row0 + NR0, NR1 - NR0)])

    pltpu.sync_copy(b_h.at[pl.ds(c * HALF, HALF)], bv)
    for h in zs:
        h.wait()
    plsc.subcore_barrier()

    sk = [1e-7 - bv[pl.ds(k * 16, 16)] for k in range(HALF // 16)]
    ebase = s * EPT

    def compute_rows(col0, g, hb, ob):
        @plsc.parallel_loop(0, EK, 1, unroll=8)
        def _(i):
            for k in range(HALF // 16):
                x = g[i, pl.ds(col0 + k * 16, 16)] + hb[i, pl.ds(k * 16, 16)]
                r = jnp.maximum(x, 0.0)
                t = jnp.exp(r + sk[k])
                ob[i, pl.ds(k * 16, 16)] = t
                ob[i, pl.ds(HALF + k * 16, 16)] = (r + 1e-7) * t

    def compute(g, hb, ob):
        @pl.when(c == 0)
        def _():
            compute_rows(0, g, hb, ob)

        @pl.when(c == 1)
        def _():
            compute_rows(HALF, g, hb, ob)

    def group(chunk0, nt):
        ls = []
        for t in range(nt):
            base = ebase + (chunk0 + t) * EK
            la = pltpu.async_copy(src_h.at[pl.ds(base, EK)], sidx[t], sem_l[t])
            lb = pltpu.async_copy(dst_h.at[pl.ds(base, EK)], didx[t], sem_l[t])
            lc = pltpu.async_copy(he_h.at[c, pl.ds(base, EK)], heb[t], sem_l[t])
            ls.append((la, lb, lc))
        gs = []
        for t in range(nt):
            for h in ls[t]:
                h.wait()
            gs.append(pltpu.async_copy(h1_h.at[sidx[t]], gat[t], sem_g[t]))
        ss = []
        for t in range(nt):
            if t >= 2:
                ss[t - 2].wait()
            gs[t].wait()
            compute(gat[t], heb[t], tb[t % 2])
            ss.append(pltpu.async_copy(tb[t % 2], acc_sh.at[didx[t]],
                                       sem_s[t], add=True))
        for t in range(max(0, nt - 2), nt):
            ss[t].wait()

    GN = ECH // 3

    def trio(q, carry):
        group(q * 3, 3)
        return carry

    lax.fori_loop(0, GN, trio, 0)
    if ECH - GN * 3:
        group(GN * 3, ECH - GN * 3)
    plsc.subcore_barrier()

    @pl.when(s < NS - 1)
    def _():
        pltpu.sync_copy(acc_sh.at[pl.ds(row0, NR0)], nd_h.at[c, pl.ds(row0, NR0)])

    @pl.when(s == NS - 1)
    def _():
        pltpu.sync_copy(acc_sh.at[pl.ds(row0, NR1)], nd_h.at[c, pl.ds(row0, NR1)])


@functools.cache
def _edge_kernel():
    return pl.kernel(
        _edge_body,
        out_type=jax.ShapeDtypeStruct((2, N, HID), F32),
        mesh=_sc_mesh(),
        scratch_types=(
            [pltpu.VMEM((EK,), jnp.int32)] * 6
            + [pltpu.VMEM((EK, HALF), F32)] * 3
            + [pltpu.VMEM((EK, HID), F32)] * 5
            + [pltpu.VMEM((HALF,), F32), pltpu.VMEM((ZR, HID), F32)]
            + [pltpu.SemaphoreType.DMA] * 9
            + [pltpu.VMEM_SHARED((N, HID), F32)]
        ),
    )


def _tc_node_body(hv_ref, nd_ref, h1p_ref, mWp_ref, mbp_ref, g_ref,
                  bt_ref, hemx_ref, hv_out, h1_out, b_out):
    den = jnp.concatenate([nd_ref[0][:, :HALF], nd_ref[1][:, :HALF]], axis=1)
    num = jnp.concatenate([nd_ref[0][:, HALF:], nd_ref[1][:, HALF:]], axis=1)
    agg = num / (den + 1e-16)
    feats = h1p_ref[...] + agg
    hv = jnp.dot(feats, mWp_ref[...], preferred_element_type=F32)
    hv = hv + mbp_ref[...] + hv_ref[...]
    hv_out[...] = hv
    mean = jnp.sum(hv, axis=0, keepdims=True) * (1.0 / N)
    xc = hv - mean
    var = jnp.sum(xc * xc, axis=0, keepdims=True) * (1.0 / N)
    h1 = g_ref[...] * xc * lax.rsqrt(var + 1e-5) + bt_ref[...]
    h1 = jnp.maximum(h1, 0.0)
    h1_out[...] = h1
    b_out[...] = jnp.max(h1, axis=0, keepdims=True) + hemx_ref[...] + 1e-7


_tc_node = pl.pallas_call(
    _tc_node_body,
    out_shape=[
        jax.ShapeDtypeStruct((N, HID), F32),
        jax.ShapeDtypeStruct((N, HID), F32),
        jax.ShapeDtypeStruct((1, HID), F32),
    ],
)


def _tc_hepre_body(eW_ref, eb_ref, ef_ref, *out_refs):
    i = pl.program_id(0)
    he_outs = out_refs[:NLAYERS]
    hemx_out = out_refs[NLAYERS]
    mx_scr = out_refs[NLAYERS + 1]

    @pl.when(i == 0)
    def _():
        mx_scr[...] = jnp.zeros((8, HID), F32)

    ef = ef_ref[...]
    for l in range(NLAYERS):
        he = jnp.dot(ef, eW_ref[l], preferred_element_type=F32)
        he = he + eb_ref[pl.ds(l, 1), :]
        he_outs[l][0] = he[:, :HALF]
        he_outs[l][1] = he[:, HALF:]
        mx_scr[pl.ds(l, 1), :] = jnp.maximum(
            mx_scr[pl.ds(l, 1), :], jnp.max(he, axis=0, keepdims=True))

    @pl.when(i == EB2 - 1)
    def _():
        hemx_out[...] = jnp.maximum(mx_scr[...], 0.0)


_tc_hepre = pl.pallas_call(
    _tc_hepre_body,
    grid=(EB2,),
    in_specs=[
        pl.BlockSpec((NLAYERS, EDIM, HID), lambda i: (0, 0, 0)),
        pl.BlockSpec((8, HID), lambda i: (0, 0)),
        pl.BlockSpec((EBR2, EDIM), lambda i: (i, 0)),
    ],
    out_specs=[pl.BlockSpec((2, EBR2, HALF), lambda i: (0, i, 0))
               for _ in range(NLAYERS)] + [pl.BlockSpec((8, HID), lambda i: (0, 0))],
    out_shape=[jax.ShapeDtypeStruct((2, E, HALF), F32)
               for _ in range(NLAYERS)] + [jax.ShapeDtypeStruct((8, HID), F32)],
    scratch_shapes=[pltpu.VMEM((8, HID), F32)],
)


def _tc_final_body(hv_ref, nd_ref, h1p_ref, mW_ref, mb_ref,
                   oW_ref, ob_ref, out_ref):
    den = jnp.concatenate([nd_ref[0][:, :HALF], nd_ref[1][:, :HALF]], axis=1)
    num = jnp.concatenate([nd_ref[0][:, HALF:], nd_ref[1][:, HALF:]], axis=1)
    agg = num / (den + 1e-16)
    feats = h1p_ref[...] + agg
    hv = jnp.dot(feats, mW_ref[...], preferred_element_type=F32)
    hv = hv + mb_ref[...] + hv_ref[...]
    hg = jnp.sum(hv, axis=0, keepdims=True) * (1.0 / N)
    out_ref[...] = jnp.dot(hg, oW_ref[...], preferred_element_type=F32) + ob_ref[...]


_tc_final = pl.pallas_call(
    _tc_final_body,
    out_shape=jax.ShapeDtypeStruct((1, HID), F32),
)


def kernel(node_feats, edge_index, edge_feats, atom_emb, gamma, beta_bn,
           eW, eb, mW, mb, out_W, out_b):
    src = edge_index[0]
    dst = edge_index[1]
    table = atom_emb.reshape(NFEAT * VOCAB, HID)
    offs = (jnp.arange(NFEAT, dtype=jnp.int32) * VOCAB)[:, None]
    idxT = node_feats.T.astype(jnp.int32) + offs
    idxT = jnp.pad(idxT, ((0, 0), (0, NPAD - N))).reshape(-1)
    hv = _atom_kernel()(table, idxT)[:N]
    hepre = _tc_hepre(eW, jnp.pad(eb, ((0, 8 - NLAYERS), (0, 0))), edge_feats)
    hes, hemx = hepre[:NLAYERS], hepre[NLAYERS]
    nd = jnp.zeros((2, N, HID), F32)
    h1p = jnp.zeros((N, HID), F32)
    mWp = jnp.zeros((HID, HID), F32)
    mbp = jnp.zeros((1, HID), F32)
    for l in range(NLAYERS):
        hv, h1, b = _tc_node(hv, nd, h1p, mWp, mbp,
                             gamma[l][None], beta_bn[l][None],
                             hemx[l][None])
        nd = _edge_kernel()(src, dst, hes[l], h1, b.reshape(HID))
        h1p = h1
        mWp, mbp = mW[l], mb[l][None]
    return _tc_final(hv, nd, h1p, mWp, mbp, out_W, out_b[None])


# speedup vs baseline: 1.0029x; 1.0029x over previous
import functools

import jax
import jax.numpy as jnp
from jax import lax
from jax.experimental import pallas as pl
from jax.experimental.pallas import tpu as pltpu
from jax.experimental.pallas import tpu_sc as plsc

N = 10000
E = 320000
HID = 128
EDIM = 16
NLAYERS = 7
VOCAB = 100
NFEAT = 9
HALF = 64
NC = 2
NS = 16
NG = 4
GK = 80
NPT = NG * GK
NPAD = NC * NS * NPT
EPT = E // NS
EK = 40
ECH = EPT // EK
NR0 = 624
NR1 = N - (NS - 1) * NR0
ZR = 16
EBR2 = 2560
EB2 = E // EBR2
F32 = jnp.float32


@functools.cache
def _sc_mesh():
    return plsc.VectorSubcoreMesh(core_axis_name="c", subcore_axis_name="s",
                                  num_cores=NC, num_subcores=NS)


def _atom_body(table_h, idx_h, out_h, idxv0, idxv1, gbuf0, gbuf1,
               idn0, idn1, idn2, idn3, zbuf,
               sem_z, sem_i0, sem_i1, sem_g0, sem_g1, sem_s0, sem_s1,
               acc_sh):
    c = lax.axis_index("c")
    s = lax.axis_index("s")
    base = (s * NC + c) * NPT
    idxv = [idxv0, idxv1]
    gbuf = [gbuf0, gbuf1]
    idn = [idn0, idn1, idn2, idn3]
    sem_i = [sem_i0, sem_i1]
    sem_g = [sem_g0, sem_g1]
    sem_s = [sem_s0, sem_s1]
    zeros16 = jnp.zeros((16,), F32)

    def zrow(i, carry):
        for k in range(HID // 16):
            zbuf[i, pl.ds(k * 16, 16)] = zeros16
        return carry

    lax.fori_loop(0, ZR, zrow, 0)
    for g in range(NG):
        for j in range(GK // 16):
            idn[g][pl.ds(j * 16, 16)] = (
                lax.iota(jnp.int32, 16) + (base + g * GK + j * 16))
    zs = [pltpu.async_copy(zbuf, acc_sh.at[pl.ds(base + i * ZR, ZR)], sem_z)
          for i in range(NPT // ZR)]
    for h in zs:
        h.wait()
    steps = [(f, g) for f in range(NFEAT) for g in range(NG)]
    hs = [None, None]
    for n, (f, g) in enumerate(steps):
        p = n % 2
        if hs[p] is not None:
            hs[p][1].wait()
        pltpu.async_copy(idx_h.at[pl.ds(f * NPAD + base + g * GK, GK)],
                         idxv[p], sem_i[p]).wait()
        gh = pltpu.async_copy(table_h.at[idxv[p]], gbuf[p], sem_g[p])
        gh.wait()
        sh = pltpu.async_copy(gbuf[p], acc_sh.at[idn[g]], sem_s[p], add=True)
        hs[p] = (gh, sh)
    for p in range(2):
        if hs[p] is not None:
            hs[p][1].wait()
    pltpu.sync_copy(acc_sh.at[pl.ds(base, NPT)], out_h.at[pl.ds(base, NPT)])


@functools.cache
def _atom_kernel():
    return pl.kernel(
        _atom_body,
        out_type=jax.ShapeDtypeStruct((NPAD, HID), F32),
        mesh=_sc_mesh(),
        scratch_types=(
            [pltpu.VMEM((GK,), jnp.int32)] * 2
            + [pltpu.VMEM((GK, HID), F32)] * 2
            + [pltpu.VMEM((GK,), jnp.int32)] * 4
            + [pltpu.VMEM((ZR, HID), F32)]
            + [pltpu.SemaphoreType.DMA] * 7
            + [pltpu.VMEM_SHARED((NPAD, HID), F32)]
        ),
    )


def _edge_body(src_h, dst_h, he_h, h1_h, b_h, nd_h,
               sidx0, sidx1, sidx2, didx0, didx1, didx2,
               heb0, heb1, heb2, gat0, gat1, gat2, tb0, tb1,
               bv, zbuf,
               sem_l0, sem_l1, sem_l2,
               sem_g0, sem_g1, sem_g2,
               sem_s0, sem_s1, sem_s2,
               acc_sh):
    sidx = [sidx0, sidx1, sidx2]
    didx = [didx0, didx1, didx2]
    heb = [heb0, heb1, heb2]
    gat = [gat0, gat1, gat2]
    tb = [tb0, tb1]
    sem_l = [sem_l0, sem_l1, sem_l2]
    sem_g = [sem_g0, sem_g1, sem_g2]
    sem_s = [sem_s0, sem_s1, sem_s2]
    c = lax.axis_index("c")
    s = lax.axis_index("s")
    zeros16 = jnp.zeros((16,), F32)

    def zrow(i, carry):
        for k in range(HID // 16):
            zbuf[i, pl.ds(k * 16, 16)] = zeros16
        return carry

    lax.fori_loop(0, ZR, zrow, 0)
    row0 = s * NR0
    zs = [pltpu.async_copy(zbuf, acc_sh.at[pl.ds(row0 + i * ZR, ZR)], sem_l0)
          for i in range(NR0 // ZR)]

    @pl.when(s == NS - 1)
    def _():
        pltpu.sync_copy(zbuf, acc_sh.at[pl.ds(row0 + NR0, NR1 - NR0)])

    pltpu.sync_copy(b_h.at[pl.ds(c * HALF, HALF)], bv)
    for h in zs:
        h.wait()
    plsc.subcore_barrier()

    sk = [1e-7 - bv[pl.ds(k * 16, 16)] for k in range(HALF // 16)]
    ebase = s * EPT

    def compute_rows(col0, g, hb, ob):
        @plsc.parallel_loop(0, EK, 1, unroll=4)
        def _(i):
            for k in range(HALF // 16):
                x = g[i, pl.ds(col0 + k * 16, 16)] + hb[i, pl.ds(k * 16, 16)]
                r = jnp.maximum(x, 0.0)
                t = jnp.exp(r + sk[k])
                ob[i, pl.ds(k * 16, 16)] = t
                ob[i, pl.ds(HALF + k * 16, 16)] = (r + 1e-7) * t

    def compute(g, hb, ob):
        @pl.when(c == 0)
        def _():
            compute_rows(0, g, hb, ob)

        @pl.when(c == 1)
        def _():
            compute_rows(HALF, g, hb, ob)

    def group(chunk0, nt):
        ls = []
        for t in range(nt):
            base = ebase + (chunk0 + t) * EK
            la = pltpu.async_copy(src_h.at[pl.ds(base, EK)], sidx[t], sem_l[t])
            lb = pltpu.async_copy(dst_h.at[pl.ds(base, EK)], didx[t], sem_l[t])
            lc = pltpu.async_copy(he_h.at[c, pl.ds(base, EK)], heb[t], sem_l[t])
            ls.append((la, lb, lc))
        gs = []
        for t in range(nt):
            for h in ls[t]:
                h.wait()
            gs.append(pltpu.async_copy(h1_h.at[sidx[t]], gat[t], sem_g[t]))
        ss = []
        for t in range(nt):
            if t >= 2:
                ss[t - 2].wait()
            gs[t].wait()
            compute(gat[t], heb[t], tb[t % 2])
            ss.append(pltpu.async_copy(tb[t % 2], acc_sh.at[didx[t]],
                                       sem_s[t], add=True))
        for t in range(max(0, nt - 2), nt):
            ss[t].wait()

    GN = ECH // 3

    def trio(q, carry):
        group(q * 3, 3)
        return carry

    lax.fori_loop(0, GN, trio, 0)
    if ECH - GN * 3:
        group(GN * 3, ECH - GN * 3)
    plsc.subcore_barrier()

    @pl.when(s < NS - 1)
    def _():
        pltpu.sync_copy(acc_sh.at[pl.ds(row0, NR0)], nd_h.at[c, pl.ds(row0, NR0)])

    @pl.when(s == NS - 1)
    def _():
        pltpu.sync_copy(acc_sh.at[pl.ds(row0, NR1)], nd_h.at[c, pl.ds(row0, NR1)])


@functools.cache
def _edge_kernel():
    return pl.kernel(
        _edge_body,
        out_type=jax.ShapeDtypeStruct((2, N, HID), F32),
        mesh=_sc_mesh(),
        scratch_types=(
            [pltpu.VMEM((EK,), jnp.int32)] * 6
            + [pltpu.VMEM((EK, HALF), F32)] * 3
            + [pltpu.VMEM((EK, HID), F32)] * 5
            + [pltpu.VMEM((HALF,), F32), pltpu.VMEM((ZR, HID), F32)]
            + [pltpu.SemaphoreType.DMA] * 9
            + [pltpu.VMEM_SHARED((N, HID), F32)]
        ),
    )


def _tc_node_body(hv_ref, nd_ref, h1p_ref, mWp_ref, mbp_ref, g_ref,
                  bt_ref, hemx_ref, hv_out, h1_out, b_out):
    den = jnp.concatenate([nd_ref[0][:, :HALF], nd_ref[1][:, :HALF]], axis=1)
    num = jnp.concatenate([nd_ref[0][:, HALF:], nd_ref[1][:, HALF:]], axis=1)
    agg = num / (den + 1e-16)
    feats = h1p_ref[...] + agg
    hv = jnp.dot(feats, mWp_ref[...], preferred_element_type=F32)
    hv = hv + mbp_ref[...] + hv_ref[...]
    hv_out[...] = hv
    mean = jnp.sum(hv, axis=0, keepdims=True) * (1.0 / N)
    xc = hv - mean
    var = jnp.sum(xc * xc, axis=0, keepdims=True) * (1.0 / N)
    h1 = g_ref[...] * xc * lax.rsqrt(var + 1e-5) + bt_ref[...]
    h1 = jnp.maximum(h1, 0.0)
    h1_out[...] = h1
    b_out[...] = jnp.max(h1, axis=0, keepdims=True) + hemx_ref[...] + 1e-7


_tc_node = pl.pallas_call(
    _tc_node_body,
    out_shape=[
        jax.ShapeDtypeStruct((N, HID), F32),
        jax.ShapeDtypeStruct((N, HID), F32),
        jax.ShapeDtypeStruct((1, HID), F32),
    ],
)


def _tc_hepre_body(eW_ref, eb_ref, ef_ref, *out_refs):
    i = pl.program_id(0)
    he_outs = out_refs[:NLAYERS]
    hemx_out = out_refs[NLAYERS]
    mx_scr = out_refs[NLAYERS + 1]

    @pl.when(i == 0)
    def _():
        mx_scr[...] = jnp.zeros((8, HID), F32)

    ef = ef_ref[...]
    for l in range(NLAYERS):
        he = jnp.dot(ef, eW_ref[l], preferred_element_type=F32)
        he = he + eb_ref[pl.ds(l, 1), :]
        he_outs[l][0] = he[:, :HALF]
        he_outs[l][1] = he[:, HALF:]
        mx_scr[pl.ds(l, 1), :] = jnp.maximum(
            mx_scr[pl.ds(l, 1), :], jnp.max(he, axis=0, keepdims=True))

    @pl.when(i == EB2 - 1)
    def _():
        hemx_out[...] = jnp.maximum(mx_scr[...], 0.0)


_tc_hepre = pl.pallas_call(
    _tc_hepre_body,
    grid=(EB2,),
    in_specs=[
        pl.BlockSpec((NLAYERS, EDIM, HID), lambda i: (0, 0, 0)),
        pl.BlockSpec((8, HID), lambda i: (0, 0)),
        pl.BlockSpec((EBR2, EDIM), lambda i: (i, 0)),
    ],
    out_specs=[pl.BlockSpec((2, EBR2, HALF), lambda i: (0, i, 0))
               for _ in range(NLAYERS)] + [pl.BlockSpec((8, HID), lambda i: (0, 0))],
    out_shape=[jax.ShapeDtypeStruct((2, E, HALF), F32)
               for _ in range(NLAYERS)] + [jax.ShapeDtypeStruct((8, HID), F32)],
    scratch_shapes=[pltpu.VMEM((8, HID), F32)],
)


def _tc_final_body(hv_ref, nd_ref, h1p_ref, mW_ref, mb_ref,
                   oW_ref, ob_ref, out_ref):
    den = jnp.concatenate([nd_ref[0][:, :HALF], nd_ref[1][:, :HALF]], axis=1)
    num = jnp.concatenate([nd_ref[0][:, HALF:], nd_ref[1][:, HALF:]], axis=1)
    agg = num / (den + 1e-16)
    feats = h1p_ref[...] + agg
    hv = jnp.dot(feats, mW_ref[...], preferred_element_type=F32)
    hv = hv + mb_ref[...] + hv_ref[...]
    hg = jnp.sum(hv, axis=0, keepdims=True) * (1.0 / N)
    out_ref[...] = jnp.dot(hg, oW_ref[...], preferred_element_type=F32) + ob_ref[...]


_tc_final = pl.pallas_call(
    _tc_final_body,
    out_shape=jax.ShapeDtypeStruct((1, HID), F32),
)


def kernel(node_feats, edge_index, edge_feats, atom_emb, gamma, beta_bn,
           eW, eb, mW, mb, out_W, out_b):
    src = edge_index[0]
    dst = edge_index[1]
    table = atom_emb.reshape(NFEAT * VOCAB, HID)
    offs = (jnp.arange(NFEAT, dtype=jnp.int32) * VOCAB)[:, None]
    idxT = node_feats.T.astype(jnp.int32) + offs
    idxT = jnp.pad(idxT, ((0, 0), (0, NPAD - N))).reshape(-1)
    hv = _atom_kernel()(table, idxT)[:N]
    hepre = _tc_hepre(eW, jnp.pad(eb, ((0, 8 - NLAYERS), (0, 0))), edge_feats)
    hes, hemx = hepre[:NLAYERS], hepre[NLAYERS]
    nd = jnp.zeros((2, N, HID), F32)
    h1p = jnp.zeros((N, HID), F32)
    mWp = jnp.zeros((HID, HID), F32)
    mbp = jnp.zeros((1, HID), F32)
    for l in range(NLAYERS):
        hv, h1, b = _tc_node(hv, nd, h1p, mWp, mbp,
                             gamma[l][None], beta_bn[l][None],
                             hemx[l][None])
        nd = _edge_kernel()(src, dst, hes[l], h1, b.reshape(HID))
        h1p = h1
        mWp, mbp = mW[l], mb[l][None]
    return _tc_final(hv, nd, h1p, mWp, mbp, out_W, out_b[None])
